# baseline (speedup 1.0000x reference)
import functools

import jax
import jax.numpy as jnp
from jax import lax
from jax.experimental import pallas as pl
from jax.experimental.pallas import tpu as pltpu
from jax.experimental.pallas import tpu_sc as plsc

N = 10000
D = 128
H = 64
C = 10
G = 64
E = 320000

NCORE = 2
NSUB = 16
NW = NCORE * NSUB
CHUNK = 128
NCHUNK = 80
EW = NCHUNK * CHUNK
EP = EW * NW
NCHUNK0 = 112
NCHUNK1 = 48
NROWS_STAGE = EP // CHUNK + NCHUNK0 - NCHUNK1
NACC = 10112
RPS = NACC // NSUB
DEGW = 16
NBUF = 4

BLK = 1000
NBLK = N // BLK


def _sc_degree(dst_rows, ones_rows, zeros_deg):
    mesh = plsc.VectorSubcoreMesh(core_axis_name="c", subcore_axis_name="s")

    @functools.partial(
        pl.kernel,
        mesh=mesh,
        out_type=jax.ShapeDtypeStruct((NCORE * NACC, DEGW), jnp.float32),
        compiler_params=pltpu.CompilerParams(use_tc_tiling_on_sc=False),
        scratch_types=[
            pltpu.VMEM((NCHUNK, CHUNK), jnp.int32),
            pltpu.VMEM((CHUNK, DEGW), jnp.float32),
            pltpu.VMEM_SHARED((NACC, DEGW), jnp.float32),
        ],
    )
    def k(dst_hbm, ones_hbm, zeros_hbm, out_hbm, dst_v, ones_v, acc):
        c = lax.axis_index("c")
        s = lax.axis_index("s")
        w = c * NSUB + s
        pltpu.sync_copy(zeros_hbm.at[pl.ds(s * RPS, RPS)],
                        acc.at[pl.ds(s * RPS, RPS)])
        pltpu.sync_copy(dst_hbm.at[pl.ds(w * NCHUNK, NCHUNK)], dst_v)
        pltpu.sync_copy(ones_hbm, ones_v)
        plsc.subcore_barrier()

        def body(j, carry):
            pltpu.sync_copy(ones_v, acc.at[dst_v.at[j]], add=True)
            return carry

        lax.fori_loop(0, NCHUNK, body, 0)
        plsc.subcore_barrier()
        pltpu.sync_copy(acc.at[pl.ds(s * RPS, RPS)],
                        out_hbm.at[pl.ds(c * NACC + s * RPS, RPS)])

    return k(dst_rows, ones_rows, zeros_deg)


def _sc_aggregate(g, src_rows, dst_rows, zeros_acc):
    mesh = plsc.VectorSubcoreMesh(core_axis_name="c", subcore_axis_name="s")

    @functools.partial(
        pl.kernel,
        mesh=mesh,
        out_type=jax.ShapeDtypeStruct((NCORE * NACC, H), jnp.float32),
        compiler_params=pltpu.CompilerParams(use_tc_tiling_on_sc=False),
        scratch_types=[
            pltpu.VMEM((NCHUNK0, CHUNK), jnp.int32),
            pltpu.VMEM((NCHUNK0, CHUNK), jnp.int32),
            [pltpu.VMEM((CHUNK, H), jnp.float32) for _ in range(NBUF)],
            [pltpu.SemaphoreType.DMA for _ in range(NBUF)],
            pltpu.VMEM_SHARED((NACC, H), jnp.float32),
        ],
    )
    def k(g_hbm, src_hbm, dst_hbm, zeros_hbm, out_hbm,
          src_v, dst_v, rows, sems, acc):
        c = lax.axis_index("c")
        s = lax.axis_index("s")
        off = c * (NCHUNK0 * NSUB) + s * jnp.where(c == 0, NCHUNK0, NCHUNK1)
        nc = jnp.where(c == 0, NCHUNK0, NCHUNK1)
        pltpu.sync_copy(zeros_hbm.at[pl.ds(s * RPS, RPS)],
                        acc.at[pl.ds(s * RPS, RPS)])
        pltpu.sync_copy(src_hbm.at[pl.ds(off, NCHUNK0)], src_v)
        pltpu.sync_copy(dst_hbm.at[pl.ds(off, NCHUNK0)], dst_v)

        for b in range(NBUF - 1):
            pltpu.async_copy(g_hbm.at[src_v.at[b]], rows[b], sems[b])
        plsc.subcore_barrier()

        def body(q, carry):
            j = q * NBUF
            for b in range(NBUF):
                jb = j + b
                pltpu.make_async_copy(g_hbm.at[src_v.at[jb]], rows[b],
                                      sems[b]).wait()
                pltpu.sync_copy(rows[b], acc.at[dst_v.at[jb]], add=True)
                nxt = jb + NBUF - 1
                bn = (b + NBUF - 1) % NBUF

                @pl.when(nxt < nc)
                def _prefetch():
                    pltpu.async_copy(g_hbm.at[src_v.at[nxt]], rows[bn],
                                     sems[bn])
            return carry

        lax.fori_loop(0, nc // NBUF, body, 0)
        plsc.subcore_barrier()
        pltpu.sync_copy(acc.at[pl.ds(s * RPS, RPS)],
                        out_hbm.at[pl.ds(c * NACC + s * RPS, RPS)])

    return k(g, src_rows, dst_rows, zeros_acc)


def _dinv(d0_ref, d1_ref):
    return lax.rsqrt(d0_ref[:, 0:1] + d1_ref[:, 0:1] + 1.0)


def _s1_body(x_ref, d0_ref, d1_ref, w1_ref, g1_ref):
    h = jnp.dot(x_ref[...], w1_ref[...], preferred_element_type=jnp.float32)
    g1_ref[...] = h * _dinv(d0_ref, d1_ref)


def _tc_stage1(x, d0, d1, W1):
    return pl.pallas_call(
        _s1_body,
        grid=(NBLK,),
        in_specs=[
            pl.BlockSpec((BLK, D), lambda i: (i, 0)),
            pl.BlockSpec((BLK, DEGW), lambda i: (i, 0)),
            pl.BlockSpec((BLK, DEGW), lambda i: (i, 0)),
            pl.BlockSpec((D, H), lambda i: (0, 0)),
        ],
        out_specs=pl.BlockSpec((BLK, H), lambda i: (i, 0)),
        out_shape=jax.ShapeDtypeStruct((N, H), jnp.float32),
    )(x, d0, d1, W1)


def _s2_body(p0_ref, p1_ref, g1_ref, d0_ref, d1_ref, w2_ref, b1_ref, g2_ref):
    dinv = _dinv(d0_ref, d1_ref)
    h = dinv * (p0_ref[...] + p1_ref[...] + g1_ref[...]) + b1_ref[...]
    h = jnp.maximum(h, 0.0)
    g2 = jnp.dot(h, w2_ref[...], preferred_element_type=jnp.float32)
    g2_ref[...] = g2 * dinv


def _tc_stage2(p0, p1, g1, d0, d1, W2, b1):
    return pl.pallas_call(
        _s2_body,
        grid=(NBLK,),
        in_specs=[
            pl.BlockSpec((BLK, H), lambda i: (i, 0)),
            pl.BlockSpec((BLK, H), lambda i: (i, 0)),
            pl.BlockSpec((BLK, H), lambda i: (i, 0)),
            pl.BlockSpec((BLK, DEGW), lambda i: (i, 0)),
            pl.BlockSpec((BLK, DEGW), lambda i: (i, 0)),
            pl.BlockSpec((H, H), lambda i: (0, 0)),
            pl.BlockSpec((1, H), lambda i: (0, 0)),
        ],
        out_specs=pl.BlockSpec((BLK, H), lambda i: (i, 0)),
        out_shape=jax.ShapeDtypeStruct((N, H), jnp.float32),
    )(p0, p1, g1, d0, d1, W2, b1)


def _s3_body(q0_ref, q1_ref, g2_ref, d0_ref, d1_ref, b2_ref, batch_ref,
             wfc_ref, bfc_ref, out_ref, sums, counts):
    i = pl.program_id(0)

    @pl.when(i == 0)
    def _init():
        sums[...] = jnp.zeros_like(sums)
        counts[...] = jnp.zeros_like(counts)

    dinv = _dinv(d0_ref, d1_ref)
    h = dinv * (q0_ref[...] + q1_ref[...] + g2_ref[...]) + b2_ref[...]
    h = jnp.maximum(h, 0.0)
    b = batch_ref[0]
    gid = lax.broadcasted_iota(jnp.int32, (G, BLK), 0)
    onehot = jnp.where(gid == b, 1.0, 0.0)
    sums[...] += jnp.dot(onehot, h, preferred_element_type=jnp.float32)
    counts[...] += jnp.broadcast_to(
        jnp.sum(onehot, axis=1, keepdims=True), (G, 128))

    @pl.when(i == NBLK - 1)
    def _fin():
        cnt = counts[:, 0:1]
        pooled = sums[...] / jnp.maximum(cnt, 1.0)
        logits = jnp.dot(pooled, wfc_ref[...],
                         preferred_element_type=jnp.float32) + bfc_ref[...]
        m = jnp.max(logits, axis=1, keepdims=True)
        z = logits - m
        lse = jnp.log(jnp.sum(jnp.exp(z), axis=1, keepdims=True))
        out_ref[...] = z - lse


def _tc_stage3(q0, q1, g2, d0, d1, b2, batch_r, Wfc, bfc):
    return pl.pallas_call(
        _s3_body,
        grid=(NBLK,),
        in_specs=[
            pl.BlockSpec((BLK, H), lambda i: (i, 0)),
            pl.BlockSpec((BLK, H), lambda i: (i, 0)),
            pl.BlockSpec((BLK, H), lambda i: (i, 0)),
            pl.BlockSpec((BLK, DEGW), lambda i: (i, 0)),
            pl.BlockSpec((BLK, DEGW), lambda i: (i, 0)),
            pl.BlockSpec((1, H), lambda i: (0, 0)),
            pl.BlockSpec((1, 1, BLK), lambda i: (i, 0, 0)),
            pl.BlockSpec((H, C), lambda i: (0, 0)),
            pl.BlockSpec((1, C), lambda i: (0, 0)),
        ],
        out_specs=pl.BlockSpec((G, C), lambda i: (0, 0)),
        out_shape=jax.ShapeDtypeStruct((G, C), jnp.float32),
        scratch_shapes=[
            pltpu.VMEM((G, H), jnp.float32),
            pltpu.VMEM((G, 128), jnp.float32),
        ],
    )(q0, q1, g2, d0, d1, b2, batch_r, Wfc, bfc)


def kernel(x, edge_index, batch, W1, b1, W2, b2, Wfc, bfc):
    src = edge_index[0]
    dst = edge_index[1]
    pad = NROWS_STAGE * CHUNK - E
    src_p = jnp.concatenate(
        [src, jnp.zeros((pad,), jnp.int32)]).reshape(NROWS_STAGE, CHUNK)
    dst_p = jnp.concatenate(
        [dst, jnp.full((pad,), N, jnp.int32)]).reshape(NROWS_STAGE, CHUNK)
    ones_deg = jnp.ones((CHUNK, DEGW), jnp.float32)
    zeros_deg = jnp.zeros((NACC, DEGW), jnp.float32)
    zeros_acc = jnp.zeros((NACC, H), jnp.float32)

    deg2 = _sc_degree(dst_p, ones_deg, zeros_deg)
    d0 = deg2[:N]
    d1 = deg2[NACC:NACC + N]

    g1 = _tc_stage1(x, d0, d1, W1)
    agg1 = _sc_aggregate(g1, src_p, dst_p, zeros_acc)
    g2 = _tc_stage2(agg1[:N], agg1[NACC:NACC + N], g1, d0, d1,
                    W2, b1.reshape(1, H))
    agg2 = _sc_aggregate(g2, src_p, dst_p, zeros_acc)

    batch_r = batch.reshape(NBLK, 1, BLK)
    return _tc_stage3(agg2[:N], agg2[NACC:NACC + N], g2, d0, d1,
                      b2.reshape(1, H), batch_r, Wfc, bfc.reshape(1, C))


# speedup vs baseline: 1.1200x; 1.1200x over previous
import functools

import jax
import jax.numpy as jnp
from jax import lax
from jax.experimental import pallas as pl
from jax.experimental.pallas import tpu as pltpu
from jax.experimental.pallas import tpu_sc as plsc

N = 10000
D = 128
H = 64
C = 10
G = 64
E = 320000

NCORE = 2
NSUB = 16
NW = NCORE * NSUB
CHUNK = 128
NCHUNK = 80
EW = NCHUNK * CHUNK
EP = EW * NW
NCHUNK0 = 48
NCHUNK1 = 112
NCMAX = max(NCHUNK0, NCHUNK1)
NROWS_STAGE = EP // CHUNK
NACC = 10112
RPS = NACC // NSUB
DEGW = 16
NBUF = 4

BLK = 1000
NBLK = N // BLK


def _sc_degree(dst_rows, ones_rows, zeros_deg):
    mesh = plsc.VectorSubcoreMesh(core_axis_name="c", subcore_axis_name="s")

    @functools.partial(
        pl.kernel,
        mesh=mesh,
        out_type=jax.ShapeDtypeStruct((NCORE * NACC, DEGW), jnp.float32),
        compiler_params=pltpu.CompilerParams(use_tc_tiling_on_sc=False),
        scratch_types=[
            pltpu.VMEM((NCHUNK, CHUNK), jnp.int32),
            pltpu.VMEM((CHUNK, DEGW), jnp.float32),
            pltpu.VMEM_SHARED((NACC, DEGW), jnp.float32),
        ],
    )
    def k(dst_hbm, ones_hbm, zeros_hbm, out_hbm, dst_v, ones_v, acc):
        c = lax.axis_index("c")
        s = lax.axis_index("s")
        w = c * NSUB + s
        pltpu.sync_copy(zeros_hbm.at[pl.ds(s * RPS, RPS)],
                        acc.at[pl.ds(s * RPS, RPS)])
        pltpu.sync_copy(dst_hbm.at[pl.ds(w * NCHUNK, NCHUNK)], dst_v)
        pltpu.sync_copy(ones_hbm, ones_v)
        plsc.subcore_barrier()

        def body(j, carry):
            pltpu.sync_copy(ones_v, acc.at[dst_v.at[j]], add=True)
            return carry

        lax.fori_loop(0, NCHUNK, body, 0)
        plsc.subcore_barrier()
        pltpu.sync_copy(acc.at[pl.ds(s * RPS, RPS)],
                        out_hbm.at[pl.ds(c * NACC + s * RPS, RPS)])

    return k(dst_rows, ones_rows, zeros_deg)


def _sc_aggregate(g, src_rows, dst_rows, zeros_acc):
    mesh = plsc.VectorSubcoreMesh(core_axis_name="c", subcore_axis_name="s")

    @functools.partial(
        pl.kernel,
        mesh=mesh,
        out_type=jax.ShapeDtypeStruct((NCORE * NACC, H), jnp.float32),
        compiler_params=pltpu.CompilerParams(use_tc_tiling_on_sc=False),
        scratch_types=[
            pltpu.VMEM((NCMAX, CHUNK), jnp.int32),
            pltpu.VMEM((NCMAX, CHUNK), jnp.int32),
            [pltpu.VMEM((CHUNK, H), jnp.float32) for _ in range(NBUF)],
            [pltpu.SemaphoreType.DMA for _ in range(NBUF)],
            pltpu.VMEM_SHARED((NACC, H), jnp.float32),
        ],
    )
    def k(g_hbm, src_hbm, dst_hbm, zeros_hbm, out_hbm,
          src_v, dst_v, rows, sems, acc):
        c = lax.axis_index("c")
        s = lax.axis_index("s")
        off = c * (NCHUNK0 * NSUB) + s * jnp.where(c == 0, NCHUNK0, NCHUNK1)
        nc = jnp.where(c == 0, NCHUNK0, NCHUNK1)
        pltpu.sync_copy(zeros_hbm.at[pl.ds(s * RPS, RPS)],
                        acc.at[pl.ds(s * RPS, RPS)])
        pltpu.sync_copy(src_hbm.at[pl.ds(off, NCMAX)], src_v)
        pltpu.sync_copy(dst_hbm.at[pl.ds(off, NCMAX)], dst_v)

        for b in range(NBUF - 1):
            pltpu.async_copy(g_hbm.at[src_v.at[b]], rows[b], sems[b])
        plsc.subcore_barrier()

        def body(q, carry):
            j = q * NBUF
            for b in range(NBUF):
                jb = j + b
                pltpu.make_async_copy(g_hbm.at[src_v.at[jb]], rows[b],
                                      sems[b]).wait()
                pltpu.sync_copy(rows[b], acc.at[dst_v.at[jb]], add=True)
                nxt = jb + NBUF - 1
                bn = (b + NBUF - 1) % NBUF

                @pl.when(nxt < nc)
                def _prefetch():
                    pltpu.async_copy(g_hbm.at[src_v.at[nxt]], rows[bn],
                                     sems[bn])
            return carry

        lax.fori_loop(0, nc // NBUF, body, 0)
        plsc.subcore_barrier()
        pltpu.sync_copy(acc.at[pl.ds(s * RPS, RPS)],
                        out_hbm.at[pl.ds(c * NACC + s * RPS, RPS)])

    return k(g, src_rows, dst_rows, zeros_acc)


def _dinv(d0_ref, d1_ref):
    return lax.rsqrt(d0_ref[:, 0:1] + d1_ref[:, 0:1] + 1.0)


def _s1_body(x_ref, d0_ref, d1_ref, w1_ref, g1_ref):
    h = jnp.dot(x_ref[...], w1_ref[...], preferred_element_type=jnp.float32)
    g1_ref[...] = h * _dinv(d0_ref, d1_ref)


def _tc_stage1(x, d0, d1, W1):
    return pl.pallas_call(
        _s1_body,
        grid=(NBLK,),
        in_specs=[
            pl.BlockSpec((BLK, D), lambda i: (i, 0)),
            pl.BlockSpec((BLK, DEGW), lambda i: (i, 0)),
            pl.BlockSpec((BLK, DEGW), lambda i: (i, 0)),
            pl.BlockSpec((D, H), lambda i: (0, 0)),
        ],
        out_specs=pl.BlockSpec((BLK, H), lambda i: (i, 0)),
        out_shape=jax.ShapeDtypeStruct((N, H), jnp.float32),
    )(x, d0, d1, W1)


def _s2_body(p0_ref, p1_ref, g1_ref, d0_ref, d1_ref, w2_ref, b1_ref, g2_ref):
    dinv = _dinv(d0_ref, d1_ref)
    h = dinv * (p0_ref[...] + p1_ref[...] + g1_ref[...]) + b1_ref[...]
    h = jnp.maximum(h, 0.0)
    g2 = jnp.dot(h, w2_ref[...], preferred_element_type=jnp.float32)
    g2_ref[...] = g2 * dinv


def _tc_stage2(p0, p1, g1, d0, d1, W2, b1):
    return pl.pallas_call(
        _s2_body,
        grid=(NBLK,),
        in_specs=[
            pl.BlockSpec((BLK, H), lambda i: (i, 0)),
            pl.BlockSpec((BLK, H), lambda i: (i, 0)),
            pl.BlockSpec((BLK, H), lambda i: (i, 0)),
            pl.BlockSpec((BLK, DEGW), lambda i: (i, 0)),
            pl.BlockSpec((BLK, DEGW), lambda i: (i, 0)),
            pl.BlockSpec((H, H), lambda i: (0, 0)),
            pl.BlockSpec((1, H), lambda i: (0, 0)),
        ],
        out_specs=pl.BlockSpec((BLK, H), lambda i: (i, 0)),
        out_shape=jax.ShapeDtypeStruct((N, H), jnp.float32),
    )(p0, p1, g1, d0, d1, W2, b1)


def _s3_body(q0_ref, q1_ref, g2_ref, d0_ref, d1_ref, b2_ref, batch_ref,
             wfc_ref, bfc_ref, out_ref, sums, counts):
    i = pl.program_id(0)

    @pl.when(i == 0)
    def _init():
        sums[...] = jnp.zeros_like(sums)
        counts[...] = jnp.zeros_like(counts)

    dinv = _dinv(d0_ref, d1_ref)
    h = dinv * (q0_ref[...] + q1_ref[...] + g2_ref[...]) + b2_ref[...]
    h = jnp.maximum(h, 0.0)
    b = batch_ref[0]
    gid = lax.broadcasted_iota(jnp.int32, (G, BLK), 0)
    onehot = jnp.where(gid == b, 1.0, 0.0)
    sums[...] += jnp.dot(onehot, h, preferred_element_type=jnp.float32)
    counts[...] += jnp.broadcast_to(
        jnp.sum(onehot, axis=1, keepdims=True), (G, 128))

    @pl.when(i == NBLK - 1)
    def _fin():
        cnt = counts[:, 0:1]
        pooled = sums[...] / jnp.maximum(cnt, 1.0)
        logits = jnp.dot(pooled, wfc_ref[...],
                         preferred_element_type=jnp.float32) + bfc_ref[...]
        m = jnp.max(logits, axis=1, keepdims=True)
        z = logits - m
        lse = jnp.log(jnp.sum(jnp.exp(z), axis=1, keepdims=True))
        out_ref[...] = z - lse


def _tc_stage3(q0, q1, g2, d0, d1, b2, batch_r, Wfc, bfc):
    return pl.pallas_call(
        _s3_body,
        grid=(NBLK,),
        in_specs=[
            pl.BlockSpec((BLK, H), lambda i: (i, 0)),
            pl.BlockSpec((BLK, H), lambda i: (i, 0)),
            pl.BlockSpec((BLK, H), lambda i: (i, 0)),
            pl.BlockSpec((BLK, DEGW), lambda i: (i, 0)),
            pl.BlockSpec((BLK, DEGW), lambda i: (i, 0)),
            pl.BlockSpec((1, H), lambda i: (0, 0)),
            pl.BlockSpec((1, 1, BLK), lambda i: (i, 0, 0)),
            pl.BlockSpec((H, C), lambda i: (0, 0)),
            pl.BlockSpec((1, C), lambda i: (0, 0)),
        ],
        out_specs=pl.BlockSpec((G, C), lambda i: (0, 0)),
        out_shape=jax.ShapeDtypeStruct((G, C), jnp.float32),
        scratch_shapes=[
            pltpu.VMEM((G, H), jnp.float32),
            pltpu.VMEM((G, 128), jnp.float32),
        ],
    )(q0, q1, g2, d0, d1, b2, batch_r, Wfc, bfc)


def kernel(x, edge_index, batch, W1, b1, W2, b2, Wfc, bfc):
    src = edge_index[0]
    dst = edge_index[1]
    pad = NROWS_STAGE * CHUNK - E
    src_p = jnp.concatenate(
        [src, jnp.zeros((pad,), jnp.int32)]).reshape(NROWS_STAGE, CHUNK)
    dst_p = jnp.concatenate(
        [dst, jnp.full((pad,), N, jnp.int32)]).reshape(NROWS_STAGE, CHUNK)
    ones_deg = jnp.ones((CHUNK, DEGW), jnp.float32)
    zeros_deg = jnp.zeros((NACC, DEGW), jnp.float32)
    zeros_acc = jnp.zeros((NACC, H), jnp.float32)

    deg2 = _sc_degree(dst_p, ones_deg, zeros_deg)
    d0 = deg2[:N]
    d1 = deg2[NACC:NACC + N]

    g1 = _tc_stage1(x, d0, d1, W1)
    agg1 = _sc_aggregate(g1, src_p, dst_p, zeros_acc)
    g2 = _tc_stage2(agg1[:N], agg1[NACC:NACC + N], g1, d0, d1,
                    W2, b1.reshape(1, H))
    agg2 = _sc_aggregate(g2, src_p, dst_p, zeros_acc)

    batch_r = batch.reshape(NBLK, 1, BLK)
    return _tc_stage3(agg2[:N], agg2[NACC:NACC + N], g2, d0, d1,
                      b2.reshape(1, H), batch_r, Wfc, bfc.reshape(1, C))


# speedup vs baseline: 1.9667x; 1.7560x over previous
import functools

import jax
import jax.numpy as jnp
from jax import lax
from jax.experimental import pallas as pl
from jax.experimental.pallas import tpu as pltpu
from jax.experimental.pallas import tpu_sc as plsc

N = 10000
D = 128
H = 64
C = 10
G = 64
E = 320000

NCORE = 2
NSUB = 16
NW = NCORE * NSUB
CHUNK = 128
NCHUNK = 80
EW = NCHUNK * CHUNK
EP = EW * NW
NCHUNK0 = 80
NCHUNK1 = 80
NCMAX = max(NCHUNK0, NCHUNK1)
NROWS_STAGE = EP // CHUNK
NACC = 10112
RPS = NACC // NSUB
DEGW = 16
NBUF = 4
HALF = H // 2

BLK = 1000
NBLK = N // BLK


def _sc_degree(dst_rows, ones_rows, zeros_deg):
    mesh = plsc.VectorSubcoreMesh(core_axis_name="c", subcore_axis_name="s")

    @functools.partial(
        pl.kernel,
        mesh=mesh,
        out_type=jax.ShapeDtypeStruct((NCORE * NACC, DEGW), jnp.float32),
        compiler_params=pltpu.CompilerParams(use_tc_tiling_on_sc=False),
        scratch_types=[
            pltpu.VMEM((NCHUNK, CHUNK), jnp.int32),
            pltpu.VMEM((CHUNK, DEGW), jnp.float32),
            pltpu.VMEM_SHARED((NACC, DEGW), jnp.float32),
        ],
    )
    def k(dst_hbm, ones_hbm, zeros_hbm, out_hbm, dst_v, ones_v, acc):
        c = lax.axis_index("c")
        s = lax.axis_index("s")
        w = c * NSUB + s
        pltpu.sync_copy(zeros_hbm.at[pl.ds(s * RPS, RPS)],
                        acc.at[pl.ds(s * RPS, RPS)])
        pltpu.sync_copy(dst_hbm.at[pl.ds(w * NCHUNK, NCHUNK)], dst_v)
        pltpu.sync_copy(ones_hbm, ones_v)
        plsc.subcore_barrier()

        def body(j, carry):
            pltpu.sync_copy(ones_v, acc.at[dst_v.at[j]], add=True)
            return carry

        lax.fori_loop(0, NCHUNK, body, 0)
        plsc.subcore_barrier()
        pltpu.sync_copy(acc.at[pl.ds(s * RPS, RPS)],
                        out_hbm.at[pl.ds(c * NACC + s * RPS, RPS)])

    return k(dst_rows, ones_rows, zeros_deg)


def _sc_aggregate(g, src_rows, dst_rows):
    mesh = plsc.VectorSubcoreMesh(core_axis_name="c", subcore_axis_name="s")

    @functools.partial(
        pl.kernel,
        mesh=mesh,
        out_type=jax.ShapeDtypeStruct((NCORE * NACC, H), jnp.float32),
        compiler_params=pltpu.CompilerParams(use_tc_tiling_on_sc=False),
        scratch_types=[
            pltpu.VMEM((NCMAX, CHUNK), jnp.int32),
            pltpu.VMEM((NCMAX, CHUNK), jnp.int32),
            [pltpu.VMEM((CHUNK, HALF), jnp.float32) for _ in range(NBUF)],
            [pltpu.SemaphoreType.DMA for _ in range(NBUF)],
            pltpu.VMEM_SHARED((NACC, HALF), jnp.float32),
            pltpu.VMEM_SHARED((N, HALF), jnp.float32),
        ],
    )
    def k(g_hbm, src_hbm, dst_hbm, out_hbm,
          src_v, dst_v, rows, sems, acc, g_spm):
        c = lax.axis_index("c")
        s = lax.axis_index("s")
        off = c * (NCHUNK0 * NSUB) + s * jnp.where(c == 0, NCHUNK0, NCHUNK1)
        nc = jnp.where(c == 0, NCHUNK0, NCHUNK1)
        pltpu.sync_copy(src_hbm.at[pl.ds(off, NCMAX)], src_v)
        pltpu.sync_copy(dst_hbm.at[pl.ds(off, NCMAX)], dst_v)

        for ph in range(2):
            if ph:
                plsc.subcore_barrier()

            @pl.when(s == 0)
            def _seed_acc():
                pltpu.sync_copy(g_hbm.at[pl.ds(0, N), pl.ds(ph * HALF, HALF)],
                                acc.at[pl.ds(0, N)])

            @pl.when(s == 1)
            def _stage_table():
                pltpu.sync_copy(g_hbm.at[pl.ds(0, N), pl.ds(ph * HALF, HALF)],
                                g_spm)

            plsc.subcore_barrier()
            for b in range(NBUF - 1):
                pltpu.async_copy(g_spm.at[src_v.at[b]], rows[b], sems[b])

            def body(q, carry):
                j = q * NBUF
                for b in range(NBUF):
                    jb = j + b
                    pltpu.make_async_copy(g_spm.at[src_v.at[jb]], rows[b],
                                          sems[b]).wait()
                    pltpu.sync_copy(rows[b], acc.at[dst_v.at[jb]], add=True)
                    nxt = jb + NBUF - 1
                    bn = (b + NBUF - 1) % NBUF

                    @pl.when(nxt < nc)
                    def _prefetch():
                        pltpu.async_copy(g_spm.at[src_v.at[nxt]], rows[bn],
                                         sems[bn])
                return carry

            lax.fori_loop(0, nc // NBUF, body, 0)
            plsc.subcore_barrier()
            pltpu.sync_copy(acc.at[pl.ds(s * RPS, RPS)],
                            out_hbm.at[pl.ds(c * NACC + s * RPS, RPS),
                                       pl.ds(ph * HALF, HALF)])

    return k(g, src_rows, dst_rows)


def _dinv(d0_ref, d1_ref):
    return lax.rsqrt(d0_ref[:, 0:1] + d1_ref[:, 0:1] + 1.0)


def _s1_body(x_ref, d0_ref, d1_ref, w1_ref, g1_ref):
    h = jnp.dot(x_ref[...], w1_ref[...], preferred_element_type=jnp.float32)
    g1_ref[...] = h * _dinv(d0_ref, d1_ref)


def _tc_stage1(x, d0, d1, W1):
    return pl.pallas_call(
        _s1_body,
        grid=(NBLK,),
        in_specs=[
            pl.BlockSpec((BLK, D), lambda i: (i, 0)),
            pl.BlockSpec((BLK, DEGW), lambda i: (i, 0)),
            pl.BlockSpec((BLK, DEGW), lambda i: (i, 0)),
            pl.BlockSpec((D, H), lambda i: (0, 0)),
        ],
        out_specs=pl.BlockSpec((BLK, H), lambda i: (i, 0)),
        out_shape=jax.ShapeDtypeStruct((N, H), jnp.float32),
    )(x, d0, d1, W1)


def _s2_body(p0_ref, p1_ref, g1_ref, d0_ref, d1_ref, w2_ref, b1_ref, g2_ref):
    dinv = _dinv(d0_ref, d1_ref)
    h = dinv * (p0_ref[...] + p1_ref[...] - g1_ref[...]) + b1_ref[...]
    h = jnp.maximum(h, 0.0)
    g2 = jnp.dot(h, w2_ref[...], preferred_element_type=jnp.float32)
    g2_ref[...] = g2 * dinv


def _tc_stage2(p0, p1, g1, d0, d1, W2, b1):
    return pl.pallas_call(
        _s2_body,
        grid=(NBLK,),
        in_specs=[
            pl.BlockSpec((BLK, H), lambda i: (i, 0)),
            pl.BlockSpec((BLK, H), lambda i: (i, 0)),
            pl.BlockSpec((BLK, H), lambda i: (i, 0)),
            pl.BlockSpec((BLK, DEGW), lambda i: (i, 0)),
            pl.BlockSpec((BLK, DEGW), lambda i: (i, 0)),
            pl.BlockSpec((H, H), lambda i: (0, 0)),
            pl.BlockSpec((1, H), lambda i: (0, 0)),
        ],
        out_specs=pl.BlockSpec((BLK, H), lambda i: (i, 0)),
        out_shape=jax.ShapeDtypeStruct((N, H), jnp.float32),
    )(p0, p1, g1, d0, d1, W2, b1)


def _s3_body(q0_ref, q1_ref, g2_ref, d0_ref, d1_ref, b2_ref, batch_ref,
             wfc_ref, bfc_ref, out_ref, sums, counts):
    i = pl.program_id(0)

    @pl.when(i == 0)
    def _init():
        sums[...] = jnp.zeros_like(sums)
        counts[...] = jnp.zeros_like(counts)

    dinv = _dinv(d0_ref, d1_ref)
    h = dinv * (q0_ref[...] + q1_ref[...] - g2_ref[...]) + b2_ref[...]
    h = jnp.maximum(h, 0.0)
    b = batch_ref[0]
    gid = lax.broadcasted_iota(jnp.int32, (G, BLK), 0)
    onehot = jnp.where(gid == b, 1.0, 0.0)
    sums[...] += jnp.dot(onehot, h, preferred_element_type=jnp.float32)
    counts[...] += jnp.broadcast_to(
        jnp.sum(onehot, axis=1, keepdims=True), (G, 128))

    @pl.when(i == NBLK - 1)
    def _fin():
        cnt = counts[:, 0:1]
        pooled = sums[...] / jnp.maximum(cnt, 1.0)
        logits = jnp.dot(pooled, wfc_ref[...],
                         preferred_element_type=jnp.float32) + bfc_ref[...]
        m = jnp.max(logits, axis=1, keepdims=True)
        z = logits - m
        lse = jnp.log(jnp.sum(jnp.exp(z), axis=1, keepdims=True))
        out_ref[...] = z - lse


def _tc_stage3(q0, q1, g2, d0, d1, b2, batch_r, Wfc, bfc):
    return pl.pallas_call(
        _s3_body,
        grid=(NBLK,),
        in_specs=[
            pl.BlockSpec((BLK, H), lambda i: (i, 0)),
            pl.BlockSpec((BLK, H), lambda i: (i, 0)),
            pl.BlockSpec((BLK, H), lambda i: (i, 0)),
            pl.BlockSpec((BLK, DEGW), lambda i: (i, 0)),
            pl.BlockSpec((BLK, DEGW), lambda i: (i, 0)),
            pl.BlockSpec((1, H), lambda i: (0, 0)),
            pl.BlockSpec((1, 1, BLK), lambda i: (i, 0, 0)),
            pl.BlockSpec((H, C), lambda i: (0, 0)),
            pl.BlockSpec((1, C), lambda i: (0, 0)),
        ],
        out_specs=pl.BlockSpec((G, C), lambda i: (0, 0)),
        out_shape=jax.ShapeDtypeStruct((G, C), jnp.float32),
        scratch_shapes=[
            pltpu.VMEM((G, H), jnp.float32),
            pltpu.VMEM((G, 128), jnp.float32),
        ],
    )(q0, q1, g2, d0, d1, b2, batch_r, Wfc, bfc)


def kernel(x, edge_index, batch, W1, b1, W2, b2, Wfc, bfc):
    src = edge_index[0]
    dst = edge_index[1]
    pad = NROWS_STAGE * CHUNK - E
    src_p = jnp.concatenate(
        [src, jnp.zeros((pad,), jnp.int32)]).reshape(NROWS_STAGE, CHUNK)
    dst_p = jnp.concatenate(
        [dst, jnp.full((pad,), N, jnp.int32)]).reshape(NROWS_STAGE, CHUNK)
    ones_deg = jnp.ones((CHUNK, DEGW), jnp.float32)
    zeros_deg = jnp.zeros((NACC, DEGW), jnp.float32)

    deg2 = _sc_degree(dst_p, ones_deg, zeros_deg)
    d0 = deg2[:N]
    d1 = deg2[NACC:NACC + N]

    g1 = _tc_stage1(x, d0, d1, W1)
    agg1 = _sc_aggregate(g1, src_p, dst_p)
    g2 = _tc_stage2(agg1[:N], agg1[NACC:NACC + N], g1, d0, d1,
                    W2, b1.reshape(1, H))
    agg2 = _sc_aggregate(g2, src_p, dst_p)

    batch_r = batch.reshape(NBLK, 1, BLK)
    return _tc_stage3(agg2[:N], agg2[NACC:NACC + N], g2, d0, d1,
                      b2.reshape(1, H), batch_r, Wfc, bfc.reshape(1, C))


# speedup vs baseline: 2.0598x; 1.0473x over previous
import functools

import jax
import jax.numpy as jnp
from jax import lax
from jax.experimental import pallas as pl
from jax.experimental.pallas import tpu as pltpu
from jax.experimental.pallas import tpu_sc as plsc

N = 10000
D = 128
H = 64
C = 10
G = 64
E = 320000

NCORE = 2
NSUB = 16
NW = NCORE * NSUB
CHUNK = 128
NCHUNK = 80
EW = NCHUNK * CHUNK
EP = EW * NW
NCHUNK0 = 80
NCHUNK1 = 80
NCMAX = max(NCHUNK0, NCHUNK1)
NROWS_STAGE = EP // CHUNK
NACC = 10112
RPS = NACC // NSUB
DEGW = 16
NBUF = 4
HALF = H // 2

BLK = 1000
NBLK = N // BLK


def _sc_degree(dst_rows, ones_rows, zeros_deg):
    mesh = plsc.VectorSubcoreMesh(core_axis_name="c", subcore_axis_name="s")

    @functools.partial(
        pl.kernel,
        mesh=mesh,
        out_type=jax.ShapeDtypeStruct((NCORE * NACC, DEGW), jnp.float32),
        compiler_params=pltpu.CompilerParams(use_tc_tiling_on_sc=False),
        scratch_types=[
            pltpu.VMEM((NCHUNK, CHUNK), jnp.int32),
            pltpu.VMEM((CHUNK, DEGW), jnp.float32),
            pltpu.VMEM_SHARED((NACC, DEGW), jnp.float32),
        ],
    )
    def k(dst_hbm, ones_hbm, zeros_hbm, out_hbm, dst_v, ones_v, acc):
        c = lax.axis_index("c")
        s = lax.axis_index("s")
        w = c * NSUB + s
        pltpu.sync_copy(zeros_hbm.at[pl.ds(s * RPS, RPS)],
                        acc.at[pl.ds(s * RPS, RPS)])
        pltpu.sync_copy(dst_hbm.at[pl.ds(w * NCHUNK, NCHUNK)], dst_v)
        pltpu.sync_copy(ones_hbm, ones_v)
        plsc.subcore_barrier()

        def body(j, carry):
            pltpu.sync_copy(ones_v, acc.at[dst_v.at[j]], add=True)
            return carry

        lax.fori_loop(0, NCHUNK, body, 0)
        plsc.subcore_barrier()
        pltpu.sync_copy(acc.at[pl.ds(s * RPS, RPS)],
                        out_hbm.at[pl.ds(c * NACC + s * RPS, RPS)])

    return k(dst_rows, ones_rows, zeros_deg)


def _sc_aggregate(g, src_rows, dst_rows):
    mesh = plsc.VectorSubcoreMesh(core_axis_name="c", subcore_axis_name="s")

    @functools.partial(
        pl.kernel,
        mesh=mesh,
        out_type=jax.ShapeDtypeStruct((NCORE * NACC, H), jnp.float32),
        compiler_params=pltpu.CompilerParams(use_tc_tiling_on_sc=False),
        scratch_types=[
            pltpu.VMEM((NCMAX, CHUNK), jnp.int32),
            pltpu.VMEM((NCMAX, CHUNK), jnp.int32),
            [pltpu.VMEM((CHUNK, HALF), jnp.float32) for _ in range(NBUF)],
            [pltpu.SemaphoreType.DMA for _ in range(NBUF)],
            [pltpu.SemaphoreType.DMA for _ in range(NBUF)],
            pltpu.VMEM_SHARED((NACC, HALF), jnp.float32),
            pltpu.VMEM_SHARED((N, HALF), jnp.float32),
        ],
    )
    def k(g_hbm, src_hbm, dst_hbm, out_hbm,
          src_v, dst_v, rows, sems, scsems, acc, g_spm):
        c = lax.axis_index("c")
        s = lax.axis_index("s")
        off = c * (NCHUNK0 * NSUB) + s * jnp.where(c == 0, NCHUNK0, NCHUNK1)
        nc = jnp.where(c == 0, NCHUNK0, NCHUNK1)
        pltpu.sync_copy(src_hbm.at[pl.ds(off, NCMAX)], src_v)
        pltpu.sync_copy(dst_hbm.at[pl.ds(off, NCMAX)], dst_v)

        for ph in range(2):
            if ph:
                plsc.subcore_barrier()

            @pl.when(s == 0)
            def _seed_acc():
                pltpu.sync_copy(g_hbm.at[pl.ds(0, N), pl.ds(ph * HALF, HALF)],
                                acc.at[pl.ds(0, N)])

            @pl.when(s == 1)
            def _stage_table():
                pltpu.sync_copy(g_hbm.at[pl.ds(0, N), pl.ds(ph * HALF, HALF)],
                                g_spm)

            plsc.subcore_barrier()
            for b in range(NBUF - 1):
                pltpu.async_copy(g_spm.at[src_v.at[b]], rows[b], sems[b])

            def body(q, carry):
                j = q * NBUF
                for b in range(NBUF):
                    jb = j + b
                    pltpu.make_async_copy(g_spm.at[src_v.at[jb]], rows[b],
                                          sems[b]).wait()
                    pltpu.async_copy(rows[b], acc.at[dst_v.at[jb]],
                                     scsems[b], add=True)
                    nxt = jb + NBUF - 1
                    bn = (b + NBUF - 1) % NBUF

                    @pl.when(nxt < nc)
                    def _prefetch():
                        @pl.when(jb >= 1)
                        def _drain_prev():
                            pltpu.make_async_copy(
                                rows[bn], acc.at[dst_v.at[jb - 1]],
                                scsems[bn]).wait()

                        pltpu.async_copy(g_spm.at[src_v.at[nxt]], rows[bn],
                                         sems[bn])
                return carry

            lax.fori_loop(0, nc // NBUF, body, 0)
            for t in range(NBUF):
                pltpu.make_async_copy(rows[t],
                                      acc.at[dst_v.at[nc - NBUF + t]],
                                      scsems[t]).wait()
            plsc.subcore_barrier()
            pltpu.sync_copy(acc.at[pl.ds(s * RPS, RPS)],
                            out_hbm.at[pl.ds(c * NACC + s * RPS, RPS),
                                       pl.ds(ph * HALF, HALF)])

    return k(g, src_rows, dst_rows)


def _dinv(d0_ref, d1_ref):
    return lax.rsqrt(d0_ref[:, 0:1] + d1_ref[:, 0:1] + 1.0)


def _s1_body(x_ref, d0_ref, d1_ref, w1_ref, g1_ref):
    h = jnp.dot(x_ref[...], w1_ref[...], preferred_element_type=jnp.float32)
    g1_ref[...] = h * _dinv(d0_ref, d1_ref)


def _tc_stage1(x, d0, d1, W1):
    return pl.pallas_call(
        _s1_body,
        grid=(NBLK,),
        in_specs=[
            pl.BlockSpec((BLK, D), lambda i: (i, 0)),
            pl.BlockSpec((BLK, DEGW), lambda i: (i, 0)),
            pl.BlockSpec((BLK, DEGW), lambda i: (i, 0)),
            pl.BlockSpec((D, H), lambda i: (0, 0)),
        ],
        out_specs=pl.BlockSpec((BLK, H), lambda i: (i, 0)),
        out_shape=jax.ShapeDtypeStruct((N, H), jnp.float32),
    )(x, d0, d1, W1)


def _s2_body(p0_ref, p1_ref, g1_ref, d0_ref, d1_ref, w2_ref, b1_ref, g2_ref):
    dinv = _dinv(d0_ref, d1_ref)
    h = dinv * (p0_ref[...] + p1_ref[...] - g1_ref[...]) + b1_ref[...]
    h = jnp.maximum(h, 0.0)
    g2 = jnp.dot(h, w2_ref[...], preferred_element_type=jnp.float32)
    g2_ref[...] = g2 * dinv


def _tc_stage2(p0, p1, g1, d0, d1, W2, b1):
    return pl.pallas_call(
        _s2_body,
        grid=(NBLK,),
        in_specs=[
            pl.BlockSpec((BLK, H), lambda i: (i, 0)),
            pl.BlockSpec((BLK, H), lambda i: (i, 0)),
            pl.BlockSpec((BLK, H), lambda i: (i, 0)),
            pl.BlockSpec((BLK, DEGW), lambda i: (i, 0)),
            pl.BlockSpec((BLK, DEGW), lambda i: (i, 0)),
            pl.BlockSpec((H, H), lambda i: (0, 0)),
            pl.BlockSpec((1, H), lambda i: (0, 0)),
        ],
        out_specs=pl.BlockSpec((BLK, H), lambda i: (i, 0)),
        out_shape=jax.ShapeDtypeStruct((N, H), jnp.float32),
    )(p0, p1, g1, d0, d1, W2, b1)


def _s3_body(q0_ref, q1_ref, g2_ref, d0_ref, d1_ref, b2_ref, batch_ref,
             wfc_ref, bfc_ref, out_ref, sums, counts):
    i = pl.program_id(0)

    @pl.when(i == 0)
    def _init():
        sums[...] = jnp.zeros_like(sums)
        counts[...] = jnp.zeros_like(counts)

    dinv = _dinv(d0_ref, d1_ref)
    h = dinv * (q0_ref[...] + q1_ref[...] - g2_ref[...]) + b2_ref[...]
    h = jnp.maximum(h, 0.0)
    b = batch_ref[0]
    gid = lax.broadcasted_iota(jnp.int32, (G, BLK), 0)
    onehot = jnp.where(gid == b, 1.0, 0.0)
    sums[...] += jnp.dot(onehot, h, preferred_element_type=jnp.float32)
    counts[...] += jnp.broadcast_to(
        jnp.sum(onehot, axis=1, keepdims=True), (G, 128))

    @pl.when(i == NBLK - 1)
    def _fin():
        cnt = counts[:, 0:1]
        pooled = sums[...] / jnp.maximum(cnt, 1.0)
        logits = jnp.dot(pooled, wfc_ref[...],
                         preferred_element_type=jnp.float32) + bfc_ref[...]
        m = jnp.max(logits, axis=1, keepdims=True)
        z = logits - m
        lse = jnp.log(jnp.sum(jnp.exp(z), axis=1, keepdims=True))
        out_ref[...] = z - lse


def _tc_stage3(q0, q1, g2, d0, d1, b2, batch_r, Wfc, bfc):
    return pl.pallas_call(
        _s3_body,
        grid=(NBLK,),
        in_specs=[
            pl.BlockSpec((BLK, H), lambda i: (i, 0)),
            pl.BlockSpec((BLK, H), lambda i: (i, 0)),
            pl.BlockSpec((BLK, H), lambda i: (i, 0)),
            pl.BlockSpec((BLK, DEGW), lambda i: (i, 0)),
            pl.BlockSpec((BLK, DEGW), lambda i: (i, 0)),
            pl.BlockSpec((1, H), lambda i: (0, 0)),
            pl.BlockSpec((1, 1, BLK), lambda i: (i, 0, 0)),
            pl.BlockSpec((H, C), lambda i: (0, 0)),
            pl.BlockSpec((1, C), lambda i: (0, 0)),
        ],
        out_specs=pl.BlockSpec((G, C), lambda i: (0, 0)),
        out_shape=jax.ShapeDtypeStruct((G, C), jnp.float32),
        scratch_shapes=[
            pltpu.VMEM((G, H), jnp.float32),
            pltpu.VMEM((G, 128), jnp.float32),
        ],
    )(q0, q1, g2, d0, d1, b2, batch_r, Wfc, bfc)


def kernel(x, edge_index, batch, W1, b1, W2, b2, Wfc, bfc):
    src = edge_index[0]
    dst = edge_index[1]
    pad = NROWS_STAGE * CHUNK - E
    src_p = jnp.concatenate(
        [src, jnp.zeros((pad,), jnp.int32)]).reshape(NROWS_STAGE, CHUNK)
    dst_p = jnp.concatenate(
        [dst, jnp.full((pad,), N, jnp.int32)]).reshape(NROWS_STAGE, CHUNK)
    ones_deg = jnp.ones((CHUNK, DEGW), jnp.float32)
    zeros_deg = jnp.zeros((NACC, DEGW), jnp.float32)

    deg2 = _sc_degree(dst_p, ones_deg, zeros_deg)
    d0 = deg2[:N]
    d1 = deg2[NACC:NACC + N]

    g1 = _tc_stage1(x, d0, d1, W1)
    agg1 = _sc_aggregate(g1, src_p, dst_p)
    g2 = _tc_stage2(agg1[:N], agg1[NACC:NACC + N], g1, d0, d1,
                    W2, b1.reshape(1, H))
    agg2 = _sc_aggregate(g2, src_p, dst_p)

    batch_r = batch.reshape(NBLK, 1, BLK)
    return _tc_stage3(agg2[:N], agg2[NACC:NACC + N], g2, d0, d1,
                      b2.reshape(1, H), batch_r, Wfc, bfc.reshape(1, C))
